# baseline (speedup 1.0000x reference)
import functools

import jax
import jax.numpy as jnp
from jax import lax
from jax.experimental import pallas as pl
from jax.experimental.pallas import tpu as pltpu
from jax.experimental.pallas import tpu_sc as plsc

CHUNK = 64
NBUF = 2


@functools.lru_cache(maxsize=None)
def _make_sc_gather(T0: int, T1: int, D: int):
    B = T0 * T1
    info = plsc.get_sparse_core_info()
    NC, NS = info.num_cores, info.num_subcores
    NW = NC * NS
    assert B % (NW * CHUNK) == 0
    b_per_w = B // NW
    n_chunks = b_per_w // CHUNK
    assert T1 % b_per_w == 0
    w_per_row = T1 // b_per_w

    mesh = plsc.VectorSubcoreMesh(core_axis_name="c", subcore_axis_name="s")

    @functools.partial(
        pl.kernel,
        mesh=mesh,
        out_type=jax.ShapeDtypeStruct((B, D), jnp.float32),
        scratch_types=[
            pltpu.VMEM((b_per_w,), jnp.int32),
            pltpu.VMEM((NBUF, CHUNK, D), jnp.float32),
            pltpu.SemaphoreType.DMA((NBUF,)),
            pltpu.SemaphoreType.DMA((NBUF,)),
        ],
    )
    def gather_kernel(idx_hbm, table_hbm, out_hbm, idx_v, bufs, gsem, wsem):
        wid = lax.axis_index("s") * NC + lax.axis_index("c")
        base = wid * b_per_w
        row = wid // w_per_row
        col = (wid % w_per_row) * b_per_w
        pltpu.sync_copy(idx_hbm.at[row, pl.ds(col, b_per_w)], idx_v)

        gathers = [
            pltpu.make_async_copy(
                table_hbm.at[idx_v.at[pl.ds(c * CHUNK, CHUNK)]],
                bufs.at[c % NBUF],
                gsem.at[c % NBUF],
            )
            for c in range(n_chunks)
        ]
        writes = [
            pltpu.make_async_copy(
                bufs.at[c % NBUF],
                out_hbm.at[pl.ds(base + c * CHUNK, CHUNK)],
                wsem.at[c % NBUF],
            )
            for c in range(n_chunks)
        ]
        for c in range(min(NBUF, n_chunks)):
            gathers[c].start()
        for c in range(n_chunks):
            gathers[c].wait()
            writes[c].start()
            if c + NBUF < n_chunks:
                writes[c].wait()
                gathers[c + NBUF].start()
        for c in range(max(0, n_chunks - NBUF), n_chunks):
            writes[c].wait()

    return gather_kernel


def kernel(tokens, W_E):
    T0, T1 = tokens.shape
    out = _make_sc_gather(T0, T1, W_E.shape[1])(tokens, W_E)
    return out.reshape(T0, T1, W_E.shape[1])


# speedup vs baseline: 1.0326x; 1.0326x over previous
import functools

import jax
import jax.numpy as jnp
from jax import lax
from jax.experimental import pallas as pl
from jax.experimental.pallas import tpu as pltpu
from jax.experimental.pallas import tpu_sc as plsc

CHUNK = 32
NBUF = 5


@functools.lru_cache(maxsize=None)
def _make_sc_gather(T0: int, T1: int, D: int):
    B = T0 * T1
    info = plsc.get_sparse_core_info()
    NC, NS = info.num_cores, info.num_subcores
    NW = NC * NS
    assert B % (NW * CHUNK) == 0
    b_per_w = B // NW
    n_chunks = b_per_w // CHUNK
    assert T1 % b_per_w == 0
    w_per_row = T1 // b_per_w

    mesh = plsc.VectorSubcoreMesh(core_axis_name="c", subcore_axis_name="s")

    @functools.partial(
        pl.kernel,
        mesh=mesh,
        out_type=jax.ShapeDtypeStruct((B, D), jnp.float32),
        scratch_types=[
            pltpu.VMEM((b_per_w,), jnp.int32),
            pltpu.VMEM((NBUF, CHUNK, D), jnp.float32),
            pltpu.SemaphoreType.DMA((NBUF,)),
            pltpu.SemaphoreType.DMA((NBUF,)),
        ],
    )
    def gather_kernel(idx_hbm, table_hbm, out_hbm, idx_v, bufs, gsem, wsem):
        wid = lax.axis_index("s") * NC + lax.axis_index("c")
        base = wid * b_per_w
        row = wid // w_per_row
        col = (wid % w_per_row) * b_per_w
        pltpu.sync_copy(idx_hbm.at[row, pl.ds(col, b_per_w)], idx_v)

        gathers = [
            pltpu.make_async_copy(
                table_hbm.at[idx_v.at[pl.ds(c * CHUNK, CHUNK)]],
                bufs.at[c % NBUF],
                gsem.at[c % NBUF],
            )
            for c in range(n_chunks)
        ]
        writes = [
            pltpu.make_async_copy(
                bufs.at[c % NBUF],
                out_hbm.at[pl.ds(base + c * CHUNK, CHUNK)],
                wsem.at[c % NBUF],
            )
            for c in range(n_chunks)
        ]
        for c in range(min(NBUF, n_chunks)):
            gathers[c].start()
        for c in range(n_chunks):
            gathers[c].wait()
            writes[c].start()
            d = c - 1
            if d >= 0 and d + NBUF < n_chunks:
                writes[d].wait()
                gathers[d + NBUF].start()
        for c in range(max(0, n_chunks - NBUF), n_chunks):
            writes[c].wait()

    return gather_kernel


def kernel(tokens, W_E):
    T0, T1 = tokens.shape
    out = _make_sc_gather(T0, T1, W_E.shape[1])(tokens, W_E)
    return out.reshape(T0, T1, W_E.shape[1])


# speedup vs baseline: 1.0383x; 1.0055x over previous
import functools

import jax
import jax.numpy as jnp
from jax import lax
from jax.experimental import pallas as pl
from jax.experimental.pallas import tpu as pltpu
from jax.experimental.pallas import tpu_sc as plsc

CHUNK = 32
NBUF = 5


@functools.lru_cache(maxsize=None)
def _make_sc_gather(T0: int, T1: int, D: int):
    B = T0 * T1
    info = plsc.get_sparse_core_info()
    NC, NS = info.num_cores, info.num_subcores
    NW = NC * NS
    assert B % (NW * CHUNK) == 0
    b_per_w = B // NW
    n_chunks = b_per_w // CHUNK
    assert T1 % b_per_w == 0
    w_per_row = T1 // b_per_w

    mesh = plsc.VectorSubcoreMesh(core_axis_name="c", subcore_axis_name="s")

    @functools.partial(
        pl.kernel,
        mesh=mesh,
        out_type=jax.ShapeDtypeStruct((B, D), jnp.float32),
        scratch_types=[
            pltpu.VMEM((b_per_w,), jnp.int32),
            pltpu.VMEM((NBUF, CHUNK, D), jnp.float32),
            pltpu.SemaphoreType.DMA((NBUF,)),
            pltpu.SemaphoreType.DMA((NBUF,)),
        ],
    )
    def gather_kernel(idx_hbm, table_hbm, out_hbm, idx_v, bufs, gsem, wsem):
        wid = lax.axis_index("s") * NC + lax.axis_index("c")
        base = wid * b_per_w
        row = wid // w_per_row
        col = (wid % w_per_row) * b_per_w
        pltpu.sync_copy(idx_hbm.at[row, pl.ds(col, b_per_w)], idx_v)

        gathers = [
            pltpu.make_async_copy(
                table_hbm.at[idx_v.at[pl.ds(c * CHUNK, CHUNK)]],
                bufs.at[c % NBUF],
                gsem.at[c % NBUF],
            )
            for c in range(n_chunks)
        ]
        writes = [
            pltpu.make_async_copy(
                bufs.at[c % NBUF],
                out_hbm.at[pl.ds(base + c * CHUNK, CHUNK)],
                wsem.at[c % NBUF],
            )
            for c in range(n_chunks)
        ]
        for c in range(min(NBUF, n_chunks)):
            gathers[c].start()
        for c in range(n_chunks):
            gathers[c].wait()
            writes[c].start()
            if c + NBUF < n_chunks:
                writes[c].wait()
                gathers[c + NBUF].start()
        for c in range(max(0, n_chunks - NBUF), n_chunks):
            writes[c].wait()

    return gather_kernel


def kernel(tokens, W_E):
    T0, T1 = tokens.shape
    out = _make_sc_gather(T0, T1, W_E.shape[1])(tokens, W_E)
    return out.reshape(T0, T1, W_E.shape[1])


# speedup vs baseline: 1.0417x; 1.0033x over previous
import functools

import jax
import jax.numpy as jnp
from jax import lax
from jax.experimental import pallas as pl
from jax.experimental.pallas import tpu as pltpu
from jax.experimental.pallas import tpu_sc as plsc

CHUNK = 32
NBUF = 5


@functools.lru_cache(maxsize=None)
def _make_sc_gather(T0: int, T1: int, D: int):
    B = T0 * T1
    info = plsc.get_sparse_core_info()
    NC, NS = info.num_cores, info.num_subcores
    NW = NC * NS
    assert B % (NW * CHUNK) == 0
    b_per_w = B // NW
    n_chunks = b_per_w // CHUNK
    assert T1 % b_per_w == 0
    w_per_row = T1 // b_per_w

    mesh = plsc.VectorSubcoreMesh(core_axis_name="c", subcore_axis_name="s")

    @functools.partial(
        pl.kernel,
        mesh=mesh,
        out_type=jax.ShapeDtypeStruct((B, D), jnp.float32),
        scratch_types=[
            pltpu.VMEM((b_per_w,), jnp.int32),
            pltpu.VMEM((NBUF, CHUNK, D), jnp.float32),
            pltpu.SemaphoreType.DMA((NBUF,)),
            pltpu.SemaphoreType.DMA((NBUF,)),
        ],
    )
    def gather_kernel(idx_hbm, table_hbm, out_hbm, idx_v, bufs, gsem, wsem):
        wid = lax.axis_index("s") * NC + lax.axis_index("c")
        base = wid * b_per_w
        row = wid // w_per_row
        col = (wid % w_per_row) * b_per_w
        pltpu.sync_copy(idx_hbm.at[row, pl.ds(col, b_per_w)], idx_v)

        gathers = [
            pltpu.make_async_copy(
                table_hbm.at[idx_v.at[pl.ds(c * CHUNK, CHUNK)]],
                bufs.at[c % NBUF],
                gsem.at[c % NBUF],
            )
            for c in range(n_chunks)
        ]
        writes = [
            pltpu.make_async_copy(
                bufs.at[c % NBUF],
                out_hbm.at[pl.ds(base + c * CHUNK, CHUNK)],
                wsem.at[c % NBUF],
            )
            for c in range(n_chunks)
        ]
        for c in range(min(NBUF, n_chunks)):
            gathers[c].start()
        for c in range(n_chunks):
            gathers[c].wait()
            writes[c].start()
            if c + NBUF < n_chunks:
                writes[c].wait()
                gathers[c + NBUF].start()
        for c in range(max(0, n_chunks - NBUF), n_chunks):
            writes[c].wait()

    return gather_kernel


def kernel(tokens, W_E):
    T0, T1 = tokens.shape
    idx = tokens.astype(jnp.int32)
    out = _make_sc_gather(T0, T1, W_E.shape[1])(idx, W_E)
    return out.reshape(T0, T1, W_E.shape[1])
